# baseline (device time: 13773 ns/iter reference)
import jax
import jax.numpy as jnp
from jax import lax
from jax.experimental import pallas as pl
from jax.experimental.pallas import tpu as pltpu

N_DEV = 8
SLICE_ROWS = 32


def kernel(x):
    _, m, n = x.shape

    def body(x_ref, out_ref, rs_buf, send_sems, recv_sems):
        my_pos = lax.axis_index("i")

        barrier_sem = pltpu.get_barrier_semaphore()
        for q in range(N_DEV):
            @pl.when(q != my_pos)
            def _():
                pl.semaphore_signal(
                    barrier_sem, inc=1,
                    device_id=(q,), device_id_type=pl.DeviceIdType.MESH,
                )
        pl.semaphore_wait(barrier_sem, N_DEV - 1)

        p1 = []
        for q in range(N_DEV):
            rdma = pltpu.make_async_remote_copy(
                src_ref=x_ref.at[0, pl.ds(q * SLICE_ROWS, SLICE_ROWS), :],
                dst_ref=rs_buf.at[my_pos],
                send_sem=send_sems.at[0, q],
                recv_sem=recv_sems.at[0, my_pos],
                device_id=(q,),
                device_id_type=pl.DeviceIdType.MESH,
            )
            p1.append(rdma)

            @pl.when(q != my_pos)
            def _():
                rdma.start()

        rs_buf[my_pos, :, :] = x_ref[0, pl.ds(my_pos * SLICE_ROWS, SLICE_ROWS), :]

        for q in range(N_DEV):
            @pl.when(q != my_pos)
            def _():
                rdma = pltpu.make_async_remote_copy(
                    src_ref=rs_buf.at[q],
                    dst_ref=rs_buf.at[q],
                    send_sem=send_sems.at[0, q],
                    recv_sem=recv_sems.at[0, q],
                    device_id=(q,),
                    device_id_type=pl.DeviceIdType.MESH,
                )
                rdma.wait_recv()

        total = rs_buf[0, :, :]
        for q in range(1, N_DEV):
            total += rs_buf[q, :, :]
        out_ref[pl.ds(my_pos * SLICE_ROWS, SLICE_ROWS), :] = total

        for q in range(N_DEV):
            rdma = pltpu.make_async_remote_copy(
                src_ref=out_ref.at[pl.ds(my_pos * SLICE_ROWS, SLICE_ROWS), :],
                dst_ref=out_ref.at[pl.ds(my_pos * SLICE_ROWS, SLICE_ROWS), :],
                send_sem=send_sems.at[1, q],
                recv_sem=recv_sems.at[1, my_pos],
                device_id=(q,),
                device_id_type=pl.DeviceIdType.MESH,
            )

            @pl.when(q != my_pos)
            def _():
                rdma.start()

        for q in range(N_DEV):
            @pl.when(q != my_pos)
            def _():
                rdma = pltpu.make_async_remote_copy(
                    src_ref=out_ref.at[pl.ds(q * SLICE_ROWS, SLICE_ROWS), :],
                    dst_ref=out_ref.at[pl.ds(q * SLICE_ROWS, SLICE_ROWS), :],
                    send_sem=send_sems.at[1, q],
                    recv_sem=recv_sems.at[1, q],
                    device_id=(q,),
                    device_id_type=pl.DeviceIdType.MESH,
                )
                rdma.wait_recv()

        for ph in range(2):
            for q in range(N_DEV):
                @pl.when(q != my_pos)
                def _():
                    rdma = pltpu.make_async_remote_copy(
                        src_ref=rs_buf.at[0],
                        dst_ref=rs_buf.at[0],
                        send_sem=send_sems.at[ph, q],
                        recv_sem=recv_sems.at[ph, q],
                        device_id=(q,),
                        device_id_type=pl.DeviceIdType.MESH,
                    )
                    rdma.wait_send()

    return pl.pallas_call(
        body,
        out_shape=jax.ShapeDtypeStruct((m, n), x.dtype),
        in_specs=[pl.BlockSpec(memory_space=pltpu.VMEM)],
        out_specs=pl.BlockSpec(memory_space=pltpu.VMEM),
        scratch_shapes=[
            pltpu.VMEM((N_DEV, SLICE_ROWS, n), x.dtype),
            pltpu.SemaphoreType.DMA((2, N_DEV)),
            pltpu.SemaphoreType.DMA((2, N_DEV)),
        ],
        compiler_params=pltpu.CompilerParams(collective_id=0),
    )(x)


# device time: 13307 ns/iter; 1.0350x vs baseline; 1.0350x over previous
import jax
import jax.numpy as jnp
from jax import lax
from jax.experimental import pallas as pl
from jax.experimental.pallas import tpu as pltpu

N_DEV = 8
MASKS = (1, 3, 4)
N_STAGES = 3
CHUNK_ROWS = (88, 88, 80)
CHUNK_STARTS = (0, 88, 176)
MAX_ROWS = max(CHUNK_ROWS)


def kernel(x):
    _, m, n = x.shape

    def body(x_ref, out_ref, comm_ref, send_sems, recv_sems):
        my_pos = lax.axis_index("i")

        barrier_sem = pltpu.get_barrier_semaphore()
        for mask in MASKS:
            pl.semaphore_signal(
                barrier_sem, inc=1,
                device_id=(my_pos ^ mask,),
                device_id_type=pl.DeviceIdType.MESH,
            )
        pl.semaphore_wait(barrier_sem, len(MASKS))

        def start_rdma(s, j):
            mask = MASKS[(j + s) % 3]
            rows = CHUNK_ROWS[j]
            start = CHUNK_STARTS[j]
            if s == 0:
                src = x_ref.at[0, pl.ds(start, rows), :]
            else:
                src = out_ref.at[pl.ds(start, rows), :]
            rdma = pltpu.make_async_remote_copy(
                src_ref=src,
                dst_ref=comm_ref.at[s, j, pl.ds(0, rows), :],
                send_sem=send_sems.at[s, j],
                recv_sem=recv_sems.at[s, j],
                device_id=(my_pos ^ mask,),
                device_id_type=pl.DeviceIdType.MESH,
            )
            rdma.start()
            return rdma

        rdmas = [start_rdma(0, j) for j in range(3)]
        for s in range(N_STAGES):
            for j in range(3):
                rows = CHUNK_ROWS[j]
                start = CHUNK_STARTS[j]
                rdmas[j].wait()
                if s == 0:
                    out_ref[pl.ds(start, rows), :] = (
                        x_ref[0, pl.ds(start, rows), :]
                        + comm_ref[s, j, pl.ds(0, rows), :]
                    )
                else:
                    out_ref[pl.ds(start, rows), :] += comm_ref[
                        s, j, pl.ds(0, rows), :
                    ]
                if s + 1 < N_STAGES:
                    rdmas[j] = start_rdma(s + 1, j)

    return pl.pallas_call(
        body,
        out_shape=jax.ShapeDtypeStruct((m, n), x.dtype),
        in_specs=[pl.BlockSpec(memory_space=pltpu.VMEM)],
        out_specs=pl.BlockSpec(memory_space=pltpu.VMEM),
        scratch_shapes=[
            pltpu.VMEM((N_STAGES, 3, MAX_ROWS, n), x.dtype),
            pltpu.SemaphoreType.DMA((N_STAGES, 3)),
            pltpu.SemaphoreType.DMA((N_STAGES, 3)),
        ],
        compiler_params=pltpu.CompilerParams(collective_id=0),
    )(x)


# device time: 12637 ns/iter; 1.0899x vs baseline; 1.0530x over previous
import jax
import jax.numpy as jnp
from jax import lax
from jax.experimental import pallas as pl
from jax.experimental.pallas import tpu as pltpu

N_DEV = 8
MASKS = (1, 3, 4)
N_STAGES = 3
CHUNK_ROWS = (48, 48, 40, 40, 40, 40)
CHUNK_STARTS = (0, 48, 96, 136, 176, 216)
N_CHUNKS = len(CHUNK_ROWS)
MAX_ROWS = max(CHUNK_ROWS)


def kernel(x):
    _, m, n = x.shape

    def body(x_ref, out_ref, comm_ref, send_sems, recv_sems):
        my_pos = lax.axis_index("i")

        barrier_sem = pltpu.get_barrier_semaphore()
        for mask in MASKS:
            pl.semaphore_signal(
                barrier_sem, inc=1,
                device_id=(my_pos ^ mask,),
                device_id_type=pl.DeviceIdType.MESH,
            )
        pl.semaphore_wait(barrier_sem, len(MASKS))

        def start_rdma(s, j):
            mask = MASKS[(j + s) % 3]
            rows = CHUNK_ROWS[j]
            start = CHUNK_STARTS[j]
            if s == 0:
                src = x_ref.at[0, pl.ds(start, rows), :]
            else:
                src = out_ref.at[pl.ds(start, rows), :]
            rdma = pltpu.make_async_remote_copy(
                src_ref=src,
                dst_ref=comm_ref.at[s, j, pl.ds(0, rows), :],
                send_sem=send_sems.at[s, j],
                recv_sem=recv_sems.at[s, j],
                device_id=(my_pos ^ mask,),
                device_id_type=pl.DeviceIdType.MESH,
            )
            rdma.start()
            return rdma

        rdmas = [start_rdma(0, j) for j in range(N_CHUNKS)]
        for s in range(N_STAGES):
            for j in range(N_CHUNKS):
                rows = CHUNK_ROWS[j]
                start = CHUNK_STARTS[j]
                rdmas[j].wait()
                if s == 0:
                    out_ref[pl.ds(start, rows), :] = (
                        x_ref[0, pl.ds(start, rows), :]
                        + comm_ref[s, j, pl.ds(0, rows), :]
                    )
                else:
                    out_ref[pl.ds(start, rows), :] += comm_ref[
                        s, j, pl.ds(0, rows), :
                    ]
                if s + 1 < N_STAGES:
                    rdmas[j] = start_rdma(s + 1, j)

    return pl.pallas_call(
        body,
        out_shape=jax.ShapeDtypeStruct((m, n), x.dtype),
        in_specs=[pl.BlockSpec(memory_space=pltpu.VMEM)],
        out_specs=pl.BlockSpec(memory_space=pltpu.VMEM),
        scratch_shapes=[
            pltpu.VMEM((N_STAGES, N_CHUNKS, MAX_ROWS, n), x.dtype),
            pltpu.SemaphoreType.DMA((N_STAGES, N_CHUNKS)),
            pltpu.SemaphoreType.DMA((N_STAGES, N_CHUNKS)),
        ],
        compiler_params=pltpu.CompilerParams(collective_id=0),
    )(x)


# device time: 12420 ns/iter; 1.1089x vs baseline; 1.0175x over previous
import jax
import jax.numpy as jnp
from jax import lax
from jax.experimental import pallas as pl
from jax.experimental.pallas import tpu as pltpu

N_DEV = 8
MASKS = (1, 3, 4)
N_STAGES = 3
CHUNK_ROWS = (24, 24, 24, 24, 24, 24, 24, 24, 16, 16, 16, 16)
CHUNK_STARTS = (0, 24, 48, 72, 96, 120, 144, 168, 192, 208, 224, 240)
N_CHUNKS = len(CHUNK_ROWS)
MAX_ROWS = max(CHUNK_ROWS)


def kernel(x):
    _, m, n = x.shape

    def body(x_ref, out_ref, comm_ref, send_sems, recv_sems):
        my_pos = lax.axis_index("i")

        barrier_sem = pltpu.get_barrier_semaphore()
        for mask in MASKS:
            pl.semaphore_signal(
                barrier_sem, inc=1,
                device_id=(my_pos ^ mask,),
                device_id_type=pl.DeviceIdType.MESH,
            )
        pl.semaphore_wait(barrier_sem, len(MASKS))

        def start_rdma(s, j):
            mask = MASKS[(j + s) % 3]
            rows = CHUNK_ROWS[j]
            start = CHUNK_STARTS[j]
            if s == 0:
                src = x_ref.at[0, pl.ds(start, rows), :]
            else:
                src = out_ref.at[pl.ds(start, rows), :]
            rdma = pltpu.make_async_remote_copy(
                src_ref=src,
                dst_ref=comm_ref.at[s, j, pl.ds(0, rows), :],
                send_sem=send_sems.at[s, j],
                recv_sem=recv_sems.at[s, j],
                device_id=(my_pos ^ mask,),
                device_id_type=pl.DeviceIdType.MESH,
            )
            rdma.start()
            return rdma

        rdmas = [start_rdma(0, j) for j in range(N_CHUNKS)]
        for s in range(N_STAGES):
            for j in range(N_CHUNKS):
                rows = CHUNK_ROWS[j]
                start = CHUNK_STARTS[j]
                rdmas[j].wait()
                if s == 0:
                    out_ref[pl.ds(start, rows), :] = (
                        x_ref[0, pl.ds(start, rows), :]
                        + comm_ref[s, j, pl.ds(0, rows), :]
                    )
                else:
                    out_ref[pl.ds(start, rows), :] += comm_ref[
                        s, j, pl.ds(0, rows), :
                    ]
                if s + 1 < N_STAGES:
                    rdmas[j] = start_rdma(s + 1, j)

    return pl.pallas_call(
        body,
        out_shape=jax.ShapeDtypeStruct((m, n), x.dtype),
        in_specs=[pl.BlockSpec(memory_space=pltpu.VMEM)],
        out_specs=pl.BlockSpec(memory_space=pltpu.VMEM),
        scratch_shapes=[
            pltpu.VMEM((N_STAGES, N_CHUNKS, MAX_ROWS, n), x.dtype),
            pltpu.SemaphoreType.DMA((N_STAGES, N_CHUNKS)),
            pltpu.SemaphoreType.DMA((N_STAGES, N_CHUNKS)),
        ],
        compiler_params=pltpu.CompilerParams(collective_id=0),
    )(x)
